# baseline (speedup 1.0000x reference)
import functools

import jax
import jax.numpy as jnp
from jax import lax
from jax.experimental import pallas as pl
from jax.experimental.pallas import tpu as pltpu
from jax.experimental.pallas import tpu_sc as plsc

_B = 4096
_L = 50
_D = 768
_N = _B * _L
_NE = 7
_NC = 2
_NS = 16
_NW = _NC * _NS
_BPW = _N // _NW
_CHUNK = 64
_NCHUNK = _BPW // _CHUNK


def _sc_gather(idx3d, emo_emb):
    mesh = plsc.VectorSubcoreMesh(core_axis_name="c", subcore_axis_name="s")

    @functools.partial(
        pl.kernel,
        mesh=mesh,
        out_type=jax.ShapeDtypeStruct((_N, _D), jnp.float32),
        scratch_types=[
            pltpu.VMEM((_NCHUNK, _CHUNK), jnp.int32),
            pltpu.VMEM((_CHUNK, _D), jnp.float32),
            pltpu.VMEM((_CHUNK, _D), jnp.float32),
            pltpu.SemaphoreType.DMA,
            pltpu.SemaphoreType.DMA,
            pltpu.SemaphoreType.DMA,
            pltpu.SemaphoreType.DMA,
        ],
    )
    def k(table_hbm, idx_hbm, out_hbm, idx_v, buf0, buf1,
          gsem0, gsem1, wsem0, wsem1):
        wid = lax.axis_index("s") * _NC + lax.axis_index("c")
        base = wid * _BPW
        pltpu.sync_copy(idx_hbm.at[wid], idx_v)

        def gather(c, buf, sem):
            pltpu.async_copy(table_hbm.at[idx_v.at[c]], buf, sem)

        def gwait(c, buf, sem):
            pltpu.make_async_copy(table_hbm.at[idx_v.at[c]], buf, sem).wait()

        def wstart(c, buf, sem):
            pltpu.async_copy(
                buf, out_hbm.at[pl.ds(base + c * _CHUNK, _CHUNK)], sem)

        def wwait(c, buf, sem):
            pltpu.make_async_copy(
                buf, out_hbm.at[pl.ds(base + c * _CHUNK, _CHUNK)], sem).wait()

        gather(0, buf0, gsem0)
        gather(1, buf1, gsem1)

        def body(g, carry):
            c = 2 * g
            gwait(c, buf0, gsem0)
            wstart(c, buf0, wsem0)
            gwait(c + 1, buf1, gsem1)
            wstart(c + 1, buf1, wsem1)
            cn = jnp.minimum(c + 2, _NCHUNK - 2)
            wwait(c, buf0, wsem0)
            gather(cn, buf0, gsem0)
            wwait(c + 1, buf1, wsem1)
            gather(cn + 1, buf1, gsem1)
            return carry

        lax.fori_loop(0, _NCHUNK // 2, body, 0)
        gwait(_NCHUNK - 2, buf0, gsem0)
        gwait(_NCHUNK - 1, buf1, gsem1)

    return k(emo_emb, idx3d)


def kernel(detect_emo, emo_emb):
    idx = detect_emo.reshape(_N).astype(jnp.int32).reshape(_NW, _NCHUNK, _CHUNK)
    out = _sc_gather(idx, emo_emb)
    return out.reshape(_B, _L, _D)


# speedup vs baseline: 2.8777x; 2.8777x over previous
import functools

import jax
import jax.numpy as jnp
from jax import lax
from jax.experimental import pallas as pl
from jax.experimental.pallas import tpu as pltpu
from jax.experimental.pallas import tpu_sc as plsc

_B = 4096
_L = 50
_D = 768
_N = _B * _L
_NE = 7
_NC = 2
_NS = 16
_NW = _NC * _NS
_BPW = _N // _NW
_NG = _BPW // 16


def _sc_lookup(idx_flat, table_flat):
    mesh = plsc.VectorSubcoreMesh(core_axis_name="c", subcore_axis_name="s")

    @functools.partial(
        pl.kernel,
        mesh=mesh,
        out_type=jax.ShapeDtypeStruct((_N * _D,), jnp.float32),
        scratch_types=[
            pltpu.VMEM((_NE * _D,), jnp.float32),
            pltpu.VMEM((_BPW,), jnp.int32),
            pltpu.SemaphoreType.DMA,
        ],
        compiler_params=pltpu.CompilerParams(needs_layout_passes=False),
    )
    def k(table_hbm, idx_hbm, out_hbm, table_v, idx_v, wsem):
        wid = lax.axis_index("s") * _NC + lax.axis_index("c")
        base = wid * _BPW
        pltpu.sync_copy(table_hbm, table_v)
        pltpu.sync_copy(idx_hbm.at[pl.ds(base, _BPW)], idx_v)

        def gbody(g, carry):
            ev = idx_v[pl.ds(g * 16, 16)]
            r0 = (base + g * 16) * _D
            for kk in range(16):
                e_off = pl.multiple_of(ev[kk] * _D, 8)
                pltpu.async_copy(
                    table_v.at[pl.ds(e_off, _D)],
                    out_hbm.at[pl.ds(r0 + kk * _D, _D)],
                    wsem)
            return carry

        lax.fori_loop(0, _NG, gbody, 0)

        def drain(j, carry):
            pltpu.make_async_copy(
                table_v.at[pl.ds(0, _D)],
                out_hbm.at[pl.ds(base * _D, _D)], wsem).wait()
            return carry

        lax.fori_loop(0, _BPW, drain, 0)

    return k(table_flat, idx_flat)


def kernel(detect_emo, emo_emb):
    idx = detect_emo.reshape(_N).astype(jnp.int32)
    out = _sc_lookup(idx, emo_emb.reshape(_NE * _D))
    return out.reshape(_B, _L, _D)
